# baseline (speedup 1.0000x reference)
import jax
import jax.numpy as jnp
from jax import lax
from jax.experimental import pallas as pl
from jax.experimental.pallas import tpu as pltpu

B = 256
HW = 28
N = 38400
CELLS = B * HW * HW
NB = 4


def _conv_body(dense_ref, ids_ref, w1r, g1r, b1r, m1r, v1r, w2r, g2r, b2r,
               m2r, v2r, w3r, g3r, b3r, m3r, v3r, out_ref):
    f32 = jnp.float32
    x = dense_ref[...]
    maskf = (ids_ref[...] >= 0).astype(f32)

    xp = jnp.pad(x, ((0, 0), (1, 1), (1, 1)))
    w1 = w1r[...]
    h1 = jnp.zeros((NB, HW, HW, 32), f32)
    for k in range(9):
        ky, kx = divmod(k, 3)
        h1 = h1 + xp[:, ky:ky + HW, kx:kx + HW][..., None] * w1[k][None, None, None, :]
    sc1 = (g1r[...] * lax.rsqrt(v1r[...] + 1e-5))
    bi1 = b1r[...] - m1r[...] * sc1
    h1 = jnp.maximum(h1 * sc1[None, None] + bi1[None, None], 0.0)
    h1 = h1 * maskf[..., None]

    h1p = jnp.pad(h1, ((0, 0), (1, 1), (1, 1), (0, 0)))
    acc = jnp.zeros((NB * HW * HW, 64), f32)
    for k in range(9):
        ky, kx = divmod(k, 3)
        a = h1p[:, ky:ky + HW, kx:kx + HW, :].reshape(NB * HW * HW, 32)
        acc = acc + jnp.dot(a, w2r[k], preferred_element_type=f32)
    sc2 = (g2r[...] * lax.rsqrt(v2r[...] + 1e-5))
    bi2 = b2r[...] - m2r[...] * sc2
    h2 = jnp.maximum(acc * sc2 + bi2, 0.0).reshape(NB, HW, HW, 64)
    h2 = h2 * maskf[..., None]

    h2r = h2.reshape(NB, 14, 2, 14, 2, 64)
    mb = jnp.broadcast_to(maskf[..., None], (NB, HW, HW, 64))
    mb = mb.reshape(NB, 14, 2, 14, 2, 64)
    mask2 = jnp.maximum(jnp.maximum(mb[:, :, 0, :, 0, :], mb[:, :, 0, :, 1, :]),
                        jnp.maximum(mb[:, :, 1, :, 0, :], mb[:, :, 1, :, 1, :]))
    acc3 = jnp.zeros((NB * 196, 64), f32)
    for k in range(4):
        dy, dx = divmod(k, 2)
        a = h2r[:, :, dy, :, dx, :].reshape(NB * 196, 64)
        acc3 = acc3 + jnp.dot(a, w3r[k], preferred_element_type=f32)
    sc3 = (g3r[...] * lax.rsqrt(v3r[...] + 1e-5))
    bi3 = b3r[...] - m3r[...] * sc3
    h3 = jnp.maximum(acc3 * sc3 + bi3, 0.0).reshape(NB, 14, 14, 64)
    out_ref[...] = h3 * mask2


def _fc_body(a_ref, w1_ref, b1_ref, w2_ref, b2_ref, out_ref):
    f32 = jnp.float32
    z1 = jnp.dot(a_ref[...], w1_ref[...], preferred_element_type=f32)
    z1 = jnp.maximum(z1 + b1_ref[...], 0.0)
    z2 = jnp.dot(z1, w2_ref[...], preferred_element_type=f32) + b2_ref[...]
    col = lax.broadcasted_iota(jnp.int32, (B, 128), 1)
    zm = jnp.where(col < 10, z2, -1e30)
    mx = jnp.max(zm, axis=1, keepdims=True)
    s = jnp.sum(jnp.exp(zm - mx), axis=1, keepdims=True)
    out_ref[...] = z2 - mx - jnp.log(s)


def _run_conv(dense, ids, w1, g1, b1, m1, v1, w2, g2, b2, m2, v2,
              w3, g3, b3, m3, v3):
    grid = B // NB
    row1 = lambda c: ((1, c), lambda i: (0, 0))
    wspec = [
        pl.BlockSpec((9, 32), lambda i: (0, 0)),
        *(pl.BlockSpec(*row1(32)) for _ in range(4)),
        pl.BlockSpec((9, 32, 64), lambda i: (0, 0, 0)),
        *(pl.BlockSpec(*row1(64)) for _ in range(4)),
        pl.BlockSpec((4, 64, 64), lambda i: (0, 0, 0)),
        *(pl.BlockSpec(*row1(64)) for _ in range(4)),
    ]
    return pl.pallas_call(
        _conv_body,
        grid=(grid,),
        in_specs=[
            pl.BlockSpec((NB, HW, HW), lambda i: (i, 0, 0)),
            pl.BlockSpec((NB, HW, HW), lambda i: (i, 0, 0)),
            *wspec,
        ],
        out_specs=pl.BlockSpec((NB, 14, 14, 64), lambda i: (i, 0, 0, 0)),
        out_shape=jax.ShapeDtypeStruct((B, 14, 14, 64), jnp.float32),
    )(dense, ids, w1.reshape(9, 32), g1.reshape(1, 32), b1.reshape(1, 32),
      m1.reshape(1, 32), v1.reshape(1, 32), w2.reshape(9, 32, 64),
      g2.reshape(1, 64), b2.reshape(1, 64), m2.reshape(1, 64),
      v2.reshape(1, 64), w3.reshape(4, 64, 64), g3.reshape(1, 64),
      b3.reshape(1, 64), m3.reshape(1, 64), v3.reshape(1, 64))


def _run_fc(h3r, fc1_w, fc1_b, fc2_w, fc2_b):
    w1 = fc1_w.reshape(64, 196, 128).transpose(1, 0, 2).reshape(196 * 64, 128)
    w2p = jnp.zeros((128, 128), jnp.float32).at[:, :10].set(fc2_w)
    b2p = jnp.zeros((1, 128), jnp.float32).at[:, :10].set(fc2_b[None, :])
    out = pl.pallas_call(
        _fc_body,
        out_shape=jax.ShapeDtypeStruct((B, 128), jnp.float32),
    )(h3r, w1, fc1_b.reshape(1, 128), w2p, b2p)
    return out[:, :10]


def kernel(features, indices, w1, g1, b1, m1, v1, w2, g2, b2, m2, v2,
           w3, g3, b3, m3, v3, fc1_w, fc1_b, fc2_w, fc2_b):
    flat = indices[:, 0] * (HW * HW) + indices[:, 1] * HW + indices[:, 2]
    dense = jnp.zeros((CELLS,), jnp.float32).at[flat].set(features[:, 0])
    ids = jnp.full((CELLS,), -1, jnp.int32).at[flat].set(jnp.arange(N, dtype=jnp.int32))
    h3r = _run_conv(dense.reshape(B, HW, HW), ids.reshape(B, HW, HW),
                    w1, g1, b1, m1, v1, w2, g2, b2, m2, v2, w3, g3, b3, m3, v3)
    return _run_fc(h3r.reshape(B, 196 * 64), fc1_w, fc1_b, fc2_w, fc2_b)


# speedup vs baseline: 1.2843x; 1.2843x over previous
import functools

import jax
import jax.numpy as jnp
from jax import lax
from jax.experimental import pallas as pl
from jax.experimental.pallas import tpu as pltpu
from jax.experimental.pallas import tpu_sc as plsc

B = 256
HW = 28
N = 38400
CELLS = B * HW * HW
NB = 4
NW = 32
CHUNK = CELLS // NW
NVEC = N // 16


def _sc_body(flat_hbm, feat_hbm, dense_hbm, ids_hbm, flat_v, feat_v,
             dense_v, ids_v, tmp_v):
    wid = lax.axis_index("s") * 2 + lax.axis_index("c")
    base = wid * CHUNK

    def zero_body(j, c):
        dense_v[pl.ds(j * 16, 16)] = jnp.zeros((16,), jnp.float32)
        ids_v[pl.ds(j * 16, 16)] = jnp.full((16,), -1, jnp.int32)
        return c

    lax.fori_loop(0, CHUNK // 16, zero_body, 0)
    pltpu.sync_copy(flat_hbm, flat_v)
    pltpu.sync_copy(feat_hbm, feat_v)

    def body(i, c):
        idx = flat_v[pl.ds(i * 16, 16)]
        rel = idx - base
        m = (rel >= 0) & (rel < CHUNK)
        lanes = lax.iota(jnp.int32, 16)
        relc = jnp.where(m, rel, 0)
        f = feat_v[pl.ds(i * 16, 16)]
        plsc.store_scatter(ids_v, [relc], lanes, mask=m)
        g = plsc.load_gather(ids_v, [relc], mask=m)
        popv = plsc.all_reduce_population_count(m & (g != lanes))
        dup = popv[0] > 0

        @pl.when(jnp.logical_not(dup))
        def _():
            plsc.store_scatter(dense_v, [relc], f, mask=m)

        @pl.when(dup)
        def _():
            for j in range(16):
                plsc.store_scatter(dense_v, [relc], f, mask=m & (lanes == j))

        return c

    lax.fori_loop(0, NVEC, body, 0)
    pltpu.sync_copy(dense_v, dense_hbm.at[pl.ds(base, CHUNK)])
    pltpu.sync_copy(ids_v, ids_hbm.at[pl.ds(base, CHUNK)])


def _densify(flat, feat):
    mesh = plsc.VectorSubcoreMesh(core_axis_name="c", subcore_axis_name="s")
    return pl.kernel(
        _sc_body,
        mesh=mesh,
        out_type=[jax.ShapeDtypeStruct((CELLS,), jnp.float32),
                  jax.ShapeDtypeStruct((CELLS,), jnp.int32)],
        scratch_types=[pltpu.VMEM((N,), jnp.int32),
                       pltpu.VMEM((N,), jnp.float32),
                       pltpu.VMEM((CHUNK,), jnp.float32),
                       pltpu.VMEM((CHUNK,), jnp.int32),
                       pltpu.VMEM((16,), jnp.int32)],
        compiler_params=pltpu.CompilerParams(needs_layout_passes=False),
    )(flat, feat)


def _conv_body(dense_ref, ids_ref, w1r, g1r, b1r, m1r, v1r, w2r, g2r, b2r,
               m2r, v2r, w3r, g3r, b3r, m3r, v3r, out_ref):
    f32 = jnp.float32
    x = dense_ref[...]
    maskf = (ids_ref[...] >= 0).astype(f32)

    xp = jnp.pad(x, ((0, 0), (1, 1), (1, 1)))
    w1 = w1r[...]
    h1 = jnp.zeros((NB, HW, HW, 32), f32)
    for k in range(9):
        ky, kx = divmod(k, 3)
        h1 = h1 + xp[:, ky:ky + HW, kx:kx + HW][..., None] * w1[k][None, None, None, :]
    sc1 = (g1r[...] * lax.rsqrt(v1r[...] + 1e-5))
    bi1 = b1r[...] - m1r[...] * sc1
    h1 = jnp.maximum(h1 * sc1[None, None] + bi1[None, None], 0.0)
    h1 = h1 * maskf[..., None]

    h1p = jnp.pad(h1, ((0, 0), (1, 1), (1, 1), (0, 0)))
    acc = jnp.zeros((NB * HW * HW, 64), f32)
    for k in range(9):
        ky, kx = divmod(k, 3)
        a = h1p[:, ky:ky + HW, kx:kx + HW, :].reshape(NB * HW * HW, 32)
        acc = acc + jnp.dot(a, w2r[k], preferred_element_type=f32)
    sc2 = (g2r[...] * lax.rsqrt(v2r[...] + 1e-5))
    bi2 = b2r[...] - m2r[...] * sc2
    h2 = jnp.maximum(acc * sc2 + bi2, 0.0).reshape(NB, HW, HW, 64)
    h2 = h2 * maskf[..., None]

    h2r = h2.reshape(NB, 14, 2, 14, 2, 64)
    mb = jnp.broadcast_to(maskf[..., None], (NB, HW, HW, 64))
    mb = mb.reshape(NB, 14, 2, 14, 2, 64)
    mask2 = jnp.maximum(jnp.maximum(mb[:, :, 0, :, 0, :], mb[:, :, 0, :, 1, :]),
                        jnp.maximum(mb[:, :, 1, :, 0, :], mb[:, :, 1, :, 1, :]))
    acc3 = jnp.zeros((NB * 196, 64), f32)
    for k in range(4):
        dy, dx = divmod(k, 2)
        a = h2r[:, :, dy, :, dx, :].reshape(NB * 196, 64)
        acc3 = acc3 + jnp.dot(a, w3r[k], preferred_element_type=f32)
    sc3 = (g3r[...] * lax.rsqrt(v3r[...] + 1e-5))
    bi3 = b3r[...] - m3r[...] * sc3
    h3 = jnp.maximum(acc3 * sc3 + bi3, 0.0).reshape(NB, 14, 14, 64)
    out_ref[...] = h3 * mask2


def _fc_body(a_ref, w1_ref, b1_ref, w2_ref, b2_ref, out_ref):
    f32 = jnp.float32
    z1 = jnp.dot(a_ref[...], w1_ref[...], preferred_element_type=f32)
    z1 = jnp.maximum(z1 + b1_ref[...], 0.0)
    z2 = jnp.dot(z1, w2_ref[...], preferred_element_type=f32) + b2_ref[...]
    col = lax.broadcasted_iota(jnp.int32, (B, 128), 1)
    zm = jnp.where(col < 10, z2, -1e30)
    mx = jnp.max(zm, axis=1, keepdims=True)
    s = jnp.sum(jnp.exp(zm - mx), axis=1, keepdims=True)
    out_ref[...] = z2 - mx - jnp.log(s)


def _run_conv(dense, ids, w1, g1, b1, m1, v1, w2, g2, b2, m2, v2,
              w3, g3, b3, m3, v3):
    grid = B // NB
    row1 = lambda c: ((1, c), lambda i: (0, 0))
    wspec = [
        pl.BlockSpec((9, 32), lambda i: (0, 0)),
        *(pl.BlockSpec(*row1(32)) for _ in range(4)),
        pl.BlockSpec((9, 32, 64), lambda i: (0, 0, 0)),
        *(pl.BlockSpec(*row1(64)) for _ in range(4)),
        pl.BlockSpec((4, 64, 64), lambda i: (0, 0, 0)),
        *(pl.BlockSpec(*row1(64)) for _ in range(4)),
    ]
    return pl.pallas_call(
        _conv_body,
        grid=(grid,),
        in_specs=[
            pl.BlockSpec((NB, HW, HW), lambda i: (i, 0, 0)),
            pl.BlockSpec((NB, HW, HW), lambda i: (i, 0, 0)),
            *wspec,
        ],
        out_specs=pl.BlockSpec((NB, 14, 14, 64), lambda i: (i, 0, 0, 0)),
        out_shape=jax.ShapeDtypeStruct((B, 14, 14, 64), jnp.float32),
    )(dense, ids, w1.reshape(9, 32), g1.reshape(1, 32), b1.reshape(1, 32),
      m1.reshape(1, 32), v1.reshape(1, 32), w2.reshape(9, 32, 64),
      g2.reshape(1, 64), b2.reshape(1, 64), m2.reshape(1, 64),
      v2.reshape(1, 64), w3.reshape(4, 64, 64), g3.reshape(1, 64),
      b3.reshape(1, 64), m3.reshape(1, 64), v3.reshape(1, 64))


def _run_fc(h3r, fc1_w, fc1_b, fc2_w, fc2_b):
    w1 = fc1_w.reshape(64, 196, 128).transpose(1, 0, 2).reshape(196 * 64, 128)
    w2p = jnp.zeros((128, 128), jnp.float32).at[:, :10].set(fc2_w)
    b2p = jnp.zeros((1, 128), jnp.float32).at[:, :10].set(fc2_b[None, :])
    out = pl.pallas_call(
        _fc_body,
        out_shape=jax.ShapeDtypeStruct((B, 128), jnp.float32),
    )(h3r, w1, fc1_b.reshape(1, 128), w2p, b2p)
    return out[:, :10]


def kernel(features, indices, w1, g1, b1, m1, v1, w2, g2, b2, m2, v2,
           w3, g3, b3, m3, v3, fc1_w, fc1_b, fc2_w, fc2_b):
    flat = indices[:, 0] * (HW * HW) + indices[:, 1] * HW + indices[:, 2]
    dense, ids = _densify(flat, features[:, 0])
    h3r = _run_conv(dense.reshape(B, HW, HW), ids.reshape(B, HW, HW),
                    w1, g1, b1, m1, v1, w2, g2, b2, m2, v2, w3, g3, b3, m3, v3)
    return _run_fc(h3r.reshape(B, 196 * 64), fc1_w, fc1_b, fc2_w, fc2_b)
